# baseline (speedup 1.0000x reference)
import jax
import jax.numpy as jnp
import numpy as np
from jax.experimental import pallas as pl
from jax.experimental.pallas import tpu as pltpu

_B, _N, _F_IN, _HID, _ITERS = 128, 200, 8, 256, 3
_SCALE = 1.0 / np.sqrt(_HID)
_J = 8


def _mm(a, b):
    return jax.lax.dot_general(
        a, b, (((1,), (0,)), ((), ())), preferred_element_type=jnp.float32
    )


def _mpnn_kernel(jets_ref, w_emb_ref, b_emb_ref,
                 w_mp0_ref, b_mp0_ref, w_mp1_ref, b_mp1_ref,
                 w_mp2_ref, b_mp2_ref,
                 w_r1_ref, b_r1_ref, w_r2_ref, b_r2_ref,
                 out_ref, a_ref):
    w_emb = w_emb_ref[...]
    b_emb = b_emb_ref[...]
    mp_params = [(w_mp0_ref[...], b_mp0_ref[...]),
                 (w_mp1_ref[...], b_mp1_ref[...]),
                 (w_mp2_ref[...], b_mp2_ref[...])]
    w_r1 = w_r1_ref[...]
    b_r1 = b_r1_ref[...]
    w_r2 = w_r2_ref[...]
    b_r2 = b_r2_ref[...]

    hs = [jnp.tanh(jax.lax.dot_general(
        jets_ref[j], w_emb, (((0,), (0,)), ((), ())),
        preferred_element_type=jnp.float32) + b_emb) for j in range(_J)]
    a_s = [None] * _J
    c2 = np.float32(_SCALE * np.log2(np.e))
    for it, (w, b) in enumerate(mp_params):
        h2 = [h * c2 for h in hs]
        logits = [jax.lax.dot_general(
            ha, hb2, (((1,), (1,)), ((), ())),
            preferred_element_type=jnp.float32)
            for ha, hb2 in zip(hs, h2)]
        ps = [jnp.exp2(l) for l in logits]
        rcp = [1.0 / jnp.sum(p, axis=-1, keepdims=True) for p in ps]
        if it == _ITERS - 1:
            a_s = [p * r for p, r in zip(ps, rcp)]
            msgs = [_mm(a, h) for a, h in zip(a_s, hs)]
        else:
            msgs = [_mm(p, h) * r for p, r, h in zip(ps, rcp, hs)]
        hs = [jnp.tanh(_mm(h, w[:_HID]) + _mm(msg, w[_HID:]) + b)
              for h, msg in zip(hs, msgs)]
    rs = [jnp.sum(jnp.tanh(_mm(h, w_r1) + b_r1), axis=0, keepdims=True)
          for h in hs]
    outs = [_mm(r, w_r2) + _N * b_r2 for r in rs]
    out_ref[...] = jnp.concatenate(outs, axis=0)
    for j in range(_J):
        a_ref[pl.ds(j * _N, _N), :] = a_s[j]


def kernel(jets, W_emb, b_emb, W_mp0, b_mp0, W_mp1, b_mp1, W_mp2, b_mp2,
           W_r1, b_r1, W_r2, b_r2):
    B, N, F_IN = jets.shape
    HID = W_emb.shape[1]

    def rep(shape):
        return pl.BlockSpec(shape, lambda b: (0,) * len(shape))

    jets_t = jnp.swapaxes(jets, 1, 2)
    b_emb2 = b_emb.reshape(1, HID)
    b_mp0_2 = b_mp0.reshape(1, HID)
    b_mp1_2 = b_mp1.reshape(1, HID)
    b_mp2_2 = b_mp2.reshape(1, HID)
    b_r1_2 = b_r1.reshape(1, HID)
    b_r2_2 = b_r2.reshape(1, HID)

    out, a = pl.pallas_call(
        _mpnn_kernel,
        grid=(B // _J,),
        in_specs=[
            pl.BlockSpec((_J, F_IN, N), lambda b: (b, 0, 0)),
            rep((F_IN, HID)), rep((1, HID)),
            rep((2 * HID, HID)), rep((1, HID)),
            rep((2 * HID, HID)), rep((1, HID)),
            rep((2 * HID, HID)), rep((1, HID)),
            rep((HID, HID)), rep((1, HID)),
            rep((HID, HID)), rep((1, HID)),
        ],
        out_specs=[
            pl.BlockSpec((_J, HID), lambda b: (b, 0)),
            pl.BlockSpec((_J * N, N), lambda b: (b, 0)),
        ],
        out_shape=[
            jax.ShapeDtypeStruct((B, HID), jnp.float32),
            jax.ShapeDtypeStruct((B * N, N), jnp.float32),
        ],
        compiler_params=pltpu.CompilerParams(
            dimension_semantics=("parallel",),
        ),
    )(jets_t, W_emb, b_emb2, W_mp0, b_mp0_2, W_mp1, b_mp1_2, W_mp2, b_mp2_2,
      W_r1, b_r1_2, W_r2, b_r2_2)
    return (out, a.reshape(B, N, N))


# speedup vs baseline: 1.1218x; 1.1218x over previous
import jax
import jax.numpy as jnp
import numpy as np
from jax.experimental import pallas as pl
from jax.experimental.pallas import tpu as pltpu

_B, _N, _F_IN, _HID, _ITERS = 128, 200, 8, 256, 3
_SCALE = 1.0 / np.sqrt(_HID)
_J = 8


def _mm(a, b):
    return jax.lax.dot_general(
        a, b, (((1,), (0,)), ((), ())), preferred_element_type=jnp.float32
    )


def _mpnn_kernel(jets_ref, w_emb_ref, b_emb_ref,
                 w_mp0_ref, b_mp0_ref, w_mp1_ref, b_mp1_ref,
                 w_mp2_ref, b_mp2_ref,
                 w_r1_ref, b_r1_ref, w_r2_ref, b_r2_ref,
                 out_ref, a_ref):
    w_emb = w_emb_ref[...]
    b_emb = b_emb_ref[...]
    mp_params = [(w_mp0_ref[...], b_mp0_ref[...]),
                 (w_mp1_ref[...], b_mp1_ref[...]),
                 (w_mp2_ref[...], b_mp2_ref[...])]
    w_r1 = w_r1_ref[...]
    b_r1 = b_r1_ref[...]
    w_r2 = w_r2_ref[...]
    b_r2 = b_r2_ref[...]

    hs = [jnp.tanh(jax.lax.dot_general(
        jets_ref[j], w_emb, (((0,), (0,)), ((), ())),
        preferred_element_type=jnp.float32) + b_emb) for j in range(_J)]
    a_s = [None] * _J
    c2 = np.float32(_SCALE * np.log2(np.e))
    for it, (w, b) in enumerate(mp_params):
        h2 = [h * c2 for h in hs]
        logits = [jax.lax.dot_general(
            ha, hb2, (((1,), (1,)), ((), ())),
            preferred_element_type=jnp.float32)
            for ha, hb2 in zip(hs, h2)]
        ps = [jnp.exp2(l) for l in logits]
        rcp = [1.0 / jnp.sum(p, axis=-1, keepdims=True) for p in ps]
        if it == _ITERS - 1:
            a_s = [p * r for p, r in zip(ps, rcp)]
            msgs = [_mm(a, h) for a, h in zip(a_s, hs)]
        else:
            msgs = [_mm(p, h) * r for p, r, h in zip(ps, rcp, hs)]
        hs = [jnp.tanh(_mm(h, w[:_HID]) + _mm(msg, w[_HID:]) + b)
              for h, msg in zip(hs, msgs)]
    rs = [jnp.sum(jnp.tanh(_mm(h, w_r1) + b_r1), axis=0, keepdims=True)
          for h in hs]
    outs = [_mm(r, w_r2) + _N * b_r2 for r in rs]
    out_ref[...] = jnp.concatenate(outs, axis=0)
    for j in range(_J):
        a_ref[j] = a_s[j]


def kernel(jets, W_emb, b_emb, W_mp0, b_mp0, W_mp1, b_mp1, W_mp2, b_mp2,
           W_r1, b_r1, W_r2, b_r2):
    B, N, F_IN = jets.shape
    HID = W_emb.shape[1]

    def rep(shape):
        return pl.BlockSpec(shape, lambda b: (0,) * len(shape))

    jets_t = jnp.swapaxes(jets, 1, 2)
    b_emb2 = b_emb.reshape(1, HID)
    b_mp0_2 = b_mp0.reshape(1, HID)
    b_mp1_2 = b_mp1.reshape(1, HID)
    b_mp2_2 = b_mp2.reshape(1, HID)
    b_r1_2 = b_r1.reshape(1, HID)
    b_r2_2 = b_r2.reshape(1, HID)

    out, a = pl.pallas_call(
        _mpnn_kernel,
        grid=(B // _J,),
        in_specs=[
            pl.BlockSpec((_J, F_IN, N), lambda b: (b, 0, 0)),
            rep((F_IN, HID)), rep((1, HID)),
            rep((2 * HID, HID)), rep((1, HID)),
            rep((2 * HID, HID)), rep((1, HID)),
            rep((2 * HID, HID)), rep((1, HID)),
            rep((HID, HID)), rep((1, HID)),
            rep((HID, HID)), rep((1, HID)),
        ],
        out_specs=[
            pl.BlockSpec((_J, HID), lambda b: (b, 0)),
            pl.BlockSpec((_J, N, N), lambda b: (b, 0, 0)),
        ],
        out_shape=[
            jax.ShapeDtypeStruct((B, HID), jnp.float32),
            jax.ShapeDtypeStruct((B, N, N), jnp.float32),
        ],
        compiler_params=pltpu.CompilerParams(
            dimension_semantics=("parallel",),
        ),
    )(jets_t, W_emb, b_emb2, W_mp0, b_mp0_2, W_mp1, b_mp1_2, W_mp2, b_mp2_2,
      W_r1, b_r1_2, W_r2, b_r2_2)
    return (out, a)
